# Initial kernel scaffold; baseline (speedup 1.0000x reference)
import jax
import jax.numpy as jnp
from jax.experimental import pallas as pl


def kernel(anchor_logits, aug_logits):
    raise NotImplementedError("write your pallas kernel here")


# speedup vs baseline: 1.3458x; 1.3458x over previous
import functools

import jax
import jax.numpy as jnp
from jax import lax
from jax.experimental import pallas as pl
from jax.experimental.pallas import tpu as pltpu
from jax.experimental.pallas import tpu_sc as plsc

_CONF = 0.015
_B, _C = 16384, 1000
_DUMP = _C
_CP = 1024
_BR = 512


def _tc_stats_body(a_ref, g_ref, tgt_ref, nll_ref):
    a = a_ref[...]
    col = lax.broadcasted_iota(jnp.int32, a.shape, 1)
    m = jnp.max(a, axis=1, keepdims=True)
    s = jnp.sum(jnp.exp(a - m), axis=1, keepdims=True)
    selected = (1.0 / s) > _CONF
    t = jnp.min(jnp.where(a >= m, col, _C), axis=1, keepdims=True)

    g = g_ref[...]
    mg = jnp.max(g, axis=1, keepdims=True)
    sg = jnp.sum(jnp.exp(g - mg), axis=1, keepdims=True)
    gt = jnp.sum(jnp.where(col == t, g, 0.0), axis=1, keepdims=True)
    nll = mg + jnp.log(sg) - gt

    tgt_ref[...] = jnp.where(selected, t, _DUMP)
    nll_ref[...] = nll


def _tc_stats(anchor, aug):
    nb = _B // _BR
    return pl.pallas_call(
        _tc_stats_body,
        grid=(nb,),
        in_specs=[
            pl.BlockSpec((_BR, _C), lambda i: (i, 0)),
            pl.BlockSpec((_BR, _C), lambda i: (i, 0)),
        ],
        out_specs=[
            pl.BlockSpec((_BR, 1), lambda i: (i, 0)),
            pl.BlockSpec((_BR, 1), lambda i: (i, 0)),
        ],
        out_shape=[
            jax.ShapeDtypeStruct((_B, 1), jnp.int32),
            jax.ShapeDtypeStruct((_B, 1), jnp.float32),
        ],
    )(anchor, aug)


def _sc_loss(tgt, nll):
    mesh = plsc.VectorSubcoreMesh(core_axis_name="c", subcore_axis_name="s")

    @functools.partial(
        pl.kernel,
        mesh=mesh,
        out_type=jax.ShapeDtypeStruct((16,), jnp.float32),
        compiler_params=pltpu.CompilerParams(needs_layout_passes=False),
        scratch_types=[
            pltpu.VMEM((_B,), jnp.int32),
            pltpu.VMEM((_B,), jnp.float32),
            pltpu.VMEM((_CP,), jnp.float32),
            pltpu.VMEM((_CP,), jnp.float32),
            pltpu.VMEM((16,), jnp.float32),
        ],
    )
    def body(tgt_hbm, nll_hbm, out_hbm, tgt_v, nll_v, counts_v, s_v, out_v):
        cid = lax.axis_index("c")
        sid = lax.axis_index("s")
        wid = sid * 2 + cid

        @pl.when(wid == 0)
        def _():
            pltpu.sync_copy(tgt_hbm, tgt_v)
            pltpu.sync_copy(nll_hbm, nll_v)
            zeros = jnp.zeros((16,), jnp.float32)
            ones = jnp.ones((16,), jnp.float32)

            def zinit(i, carry):
                counts_v[pl.ds(i * 16, 16)] = zeros
                s_v[pl.ds(i * 16, 16)] = zeros
                return carry

            lax.fori_loop(0, _CP // 16, zinit, 0)

            def scat(i, carry):
                idx = tgt_v[pl.ds(i * 16, 16)]
                val = nll_v[pl.ds(i * 16, 16)]
                plsc.addupdate_scatter(counts_v, [idx], ones)
                plsc.addupdate_scatter(s_v, [idx], val)
                return carry

            lax.fori_loop(0, _B // 16, scat, 0)

            lane = lax.iota(jnp.int32, 16)

            def red(i, carry):
                num, den = carry
                c = counts_v[pl.ds(i * 16, 16)]
                sv = s_v[pl.ds(i * 16, 16)]
                valid = (i * 16 + lane) < _C
                present = jnp.logical_and(valid, c > 0.0)
                contrib = jnp.where(present, sv / jnp.maximum(c, 1.0), 0.0)
                num = num + contrib
                den = den + jnp.where(present, ones, zeros)
                return num, den

            num, den = lax.fori_loop(0, _CP // 16, red, (zeros, zeros))
            num_v = jnp.broadcast_to(jnp.sum(num), (16,))
            den_v = jnp.broadcast_to(jnp.sum(den), (16,))
            out_v[...] = num_v / den_v
            pltpu.sync_copy(out_v, out_hbm)

    return body(tgt, nll)


def kernel(anchor_logits, aug_logits):
    tgt, nll = _tc_stats(anchor_logits, aug_logits)
    out = _sc_loss(tgt.reshape(_B), nll.reshape(_B))
    return out[0]
